# baseline (speedup 1.0000x reference)
import functools

import jax
import jax.numpy as jnp
from jax import lax
from jax.experimental import pallas as pl
from jax.experimental.pallas import tpu as pltpu
from jax.experimental.pallas import tpu_sc as plsc

NN = 10000
NP = 10240
NE = 160000
NKF = 2
NG = 8
BN_S = 1.0 / (1.0 + 1e-5) ** 0.5
BE = 4000
BM = 1024
F32 = jnp.float32
I32 = jnp.int32


def _pad_cols(x, w):
    return jnp.pad(x, ((0, 0), (0, w - x.shape[1]))) if x.shape[1] != w else x


def _pad_rows(x, r):
    return jnp.pad(x, ((0, r - x.shape[0]), (0, 0))) if x.shape[0] != r else x


def _sc_gather_multi(specs):
    info = plsc.get_sparse_core_info()
    nc, ns = info.num_cores, info.num_subcores
    nw = nc * ns
    nsp = len(specs)
    plan = []
    for table, idx in specs:
        w = table.shape[1]
        b = idx.shape[0]
        assert b % (8 * nw) == 0 and w % 128 == 0
        nb = b // nw
        bufb = 460000 // nsp - nb * 4
        best = None
        for nbuf in (5, 4, 3, 2, 1):
            for c in range(nb, 7, -1):
                if nb % c or c % 8 or (nb // c) % nbuf:
                    continue
                if nbuf * c * w * 4 <= bufb:
                    best = (nb, c, nbuf)
                    break
            if best:
                break
        assert best, (nb, w)
        plan.append(best)

    mesh = plsc.VectorSubcoreMesh(core_axis_name="c", subcore_axis_name="s")
    out_types = [jax.ShapeDtypeStruct(idx.shape + (t.shape[1],), F32)
                 for t, idx in specs]
    scratch = []
    for (t, _), (nb, c, nbuf) in zip(specs, plan):
        scratch.append(pltpu.VMEM((nb,), I32))
        for _ in range(nbuf):
            scratch.append(pltpu.VMEM((c, t.shape[1]), F32))
            scratch.append(pltpu.SemaphoreType.DMA)
            scratch.append(pltpu.SemaphoreType.DMA)

    @functools.partial(pl.kernel, mesh=mesh, out_type=out_types,
                       scratch_types=scratch)
    def k(*refs):
        tables = refs[0:2 * nsp:2]
        idxs = refs[1:2 * nsp:2]
        outs = refs[2 * nsp:3 * nsp]
        sc = list(refs[3 * nsp:])
        wid = lax.axis_index("s") * nc + lax.axis_index("c")
        p = 0
        for j in range(nsp):
            nb, c, nbuf = plan[j]
            tab, out = tables[j], outs[j]
            idxf = sc[p]; p += 1
            rows, sg, sw = [], [], []
            for _ in range(nbuf):
                rows.append(sc[p]); sg.append(sc[p + 1]); sw.append(sc[p + 2])
                p += 3
            ngroups = (nb // c) // nbuf
            base = wid * nb
            pltpu.sync_copy(idxs[j].at[pl.ds(base, nb)], idxf)
            for b in range(nbuf):
                pltpu.async_copy(tab.at[idxf.at[pl.ds(b * c, c)]],
                                 rows[b], sg[b])

            def group(cg, _, c=c, nbuf=nbuf, ngroups=ngroups, base=base,
                      tab=tab, out=out, idxf=idxf, rows=rows, sg=sg, sw=sw):
                for b in range(nbuf):
                    loc = (cg * nbuf + b) * c
                    pltpu.make_async_copy(
                        tab.at[idxf.at[pl.ds(loc, c)]], rows[b], sg[b]).wait()
                    pltpu.async_copy(rows[b], out.at[pl.ds(base + loc, c)],
                                     sw[b])

                    @pl.when(cg < ngroups - 1)
                    def _(b=b, loc=loc):
                        pltpu.make_async_copy(
                            rows[b], out.at[pl.ds(base + loc, c)],
                            sw[b]).wait()
                        nxt = loc + nbuf * c
                        pltpu.async_copy(tab.at[idxf.at[pl.ds(nxt, c)]],
                                         rows[b], sg[b])
                return 0

            lax.fori_loop(0, ngroups, group, 0)
            for b in range(nbuf):
                pltpu.make_async_copy(rows[b], out.at[pl.ds(base, c)],
                                      sw[b]).wait()

    flat = []
    for t, idx in specs:
        flat += [t, idx]
    out = k(*flat)
    return list(out) if isinstance(out, (tuple, list)) else [out]


def _sc_gather_add(t1, i1, t2, i2):
    info = plsc.get_sparse_core_info()
    nc, ns = info.num_cores, info.num_subcores
    nw = nc * ns
    w = t1.shape[1]
    b = i1.shape[0]
    assert b % (8 * nw) == 0 and w % 128 == 0
    nb = b // nw
    bufb = 460000 - 2 * nb * 4
    best = None
    for nbuf in (5, 4, 3, 2, 1):
        for c in range(nb, 7, -1):
            if nb % c or c % 8 or (nb // c) % nbuf:
                continue
            if nbuf * 2 * c * w * 4 <= bufb:
                best = (c, nbuf)
                break
        if best:
            break
    assert best, (nb, w)
    c, nbuf = best
    ngroups = (nb // c) // nbuf

    mesh = plsc.VectorSubcoreMesh(core_axis_name="c", subcore_axis_name="s")
    scratch = [pltpu.VMEM((nb,), I32), pltpu.VMEM((nb,), I32)]
    for _ in range(nbuf):
        scratch.append(pltpu.VMEM((c, w), F32))
        scratch.append(pltpu.VMEM((c, w), F32))
        scratch.append(pltpu.SemaphoreType.DMA)
        scratch.append(pltpu.SemaphoreType.DMA)

    @functools.partial(pl.kernel, mesh=mesh,
                       out_type=jax.ShapeDtypeStruct((b, w), F32),
                       scratch_types=scratch)
    def k(t1r, i1r, t2r, i2r, out, ix1, ix2, *sc):
        wid = lax.axis_index("s") * nc + lax.axis_index("c")
        base = wid * nb
        ra, rb, sg, sw = [], [], [], []
        for bi in range(nbuf):
            ra.append(sc[4 * bi]); rb.append(sc[4 * bi + 1])
            sg.append(sc[4 * bi + 2]); sw.append(sc[4 * bi + 3])
        pltpu.sync_copy(i1r.at[pl.ds(base, nb)], ix1)
        pltpu.sync_copy(i2r.at[pl.ds(base, nb)], ix2)
        for bi in range(nbuf):
            pltpu.async_copy(t1r.at[ix1.at[pl.ds(bi * c, c)]], ra[bi], sg[bi])
            pltpu.async_copy(t2r.at[ix2.at[pl.ds(bi * c, c)]], rb[bi], sg[bi])

        def group(cg, _):
            for bi in range(nbuf):
                loc = (cg * nbuf + bi) * c
                pltpu.make_async_copy(
                    t1r.at[ix1.at[pl.ds(loc, c)]], ra[bi], sg[bi]).wait()
                pltpu.make_async_copy(
                    t2r.at[ix2.at[pl.ds(loc, c)]], rb[bi], sg[bi]).wait()

                def addrow(r, _, bi=bi):
                    for jj in range(w // 16):
                        ra[bi][r, pl.ds(jj * 16, 16)] = (
                            ra[bi][r, pl.ds(jj * 16, 16)]
                            + rb[bi][r, pl.ds(jj * 16, 16)])
                    return 0

                lax.fori_loop(0, c, addrow, 0)
                pltpu.async_copy(ra[bi], out.at[pl.ds(base + loc, c)], sw[bi])

                @pl.when(cg < ngroups - 1)
                def _(bi=bi, loc=loc):
                    pltpu.make_async_copy(
                        ra[bi], out.at[pl.ds(base + loc, c)], sw[bi]).wait()
                    nxt = loc + nbuf * c
                    pltpu.async_copy(t1r.at[ix1.at[pl.ds(nxt, c)]],
                                     ra[bi], sg[bi])
                    pltpu.async_copy(t2r.at[ix2.at[pl.ds(nxt, c)]],
                                     rb[bi], sg[bi])
            return 0

        lax.fori_loop(0, ngroups, group, 0)
        for bi in range(nbuf):
            pltpu.make_async_copy(ra[bi], out.at[pl.ds(base, c)],
                                  sw[bi]).wait()

    return k(t1, i1, t2, i2)


def _conv_kernel(*refs, h1, h2, wo, first):
    if first:
        g, dst, wp, bp, wc, b1, w2, b2, out, pdo, crow, cdst = refs
        pdiff = g[:, 32:48]
    else:
        g, pdr, dst, wp, bp, wc, b1, w2, b2, out, crow, cdst = refs
        pdiff = pdr[...]
    i = pl.program_id(0)

    @pl.when(i == 0)
    def _():
        cdst[0] = -1

    pf = jnp.maximum(jnp.dot(pdiff, wp[...],
                             preferred_element_type=F32) + bp[...], 0.) * BN_S
    l1 = g[:, :h1] + jnp.dot(pf, wc[...],
                             preferred_element_type=F32) + b1[...]
    l1 = jnp.maximum(l1, 0.) * BN_S
    m = jnp.maximum(jnp.dot(l1, w2[...],
                            preferred_element_type=F32) + b2[...], 0.) * BN_S
    d = dst[...]
    s = 1
    while s < BE:
        dsh = jnp.concatenate([d[s:], jnp.full((s, 1), -2, I32)], axis=0)
        msh = jnp.concatenate([m[s:], jnp.zeros((s, h2), F32)], axis=0)
        m = jnp.where(dsh == d, jnp.maximum(m, msh), m)
        s *= 2
    m = jnp.where(d == cdst[0], jnp.maximum(m, crow[0:1, :]), m)
    cdst[0] = d[0, 0]
    crow[0:1, :] = m[0:1, :]
    if wo > h2:
        out[...] = jnp.concatenate([m, jnp.zeros((BE, wo - h2), F32)], axis=1)
    else:
        out[...] = m
    if first:
        pdo[...] = pdiff


def _conv_fused(g, pdiff, dst2, wp, bp, wc, b1, w2, b2, first):
    wt = g.shape[1]
    h1 = wc.shape[1]
    h2 = w2.shape[1]
    wo = max(h2, 128)
    nb = NE // BE
    rev = lambda i: (nb - 1 - i, 0)
    fix = lambda i: (0, 0)
    in_specs = [pl.BlockSpec((BE, wt), rev)]
    args = [g]
    if not first:
        in_specs.append(pl.BlockSpec((BE, 16), rev))
        args.append(pdiff)
    in_specs += [
        pl.BlockSpec((BE, 1), rev),
        pl.BlockSpec((16, 16), fix),
        pl.BlockSpec((1, 16), fix),
        pl.BlockSpec((16, h1), fix),
        pl.BlockSpec((1, h1), fix),
        pl.BlockSpec((h1, h2), fix),
        pl.BlockSpec((1, h2), fix),
    ]
    args += [dst2, wp, bp.reshape(1, -1), wc, b1.reshape(1, -1), w2,
             b2.reshape(1, -1)]
    out_specs = [pl.BlockSpec((BE, wo), rev)]
    out_shape = [jax.ShapeDtypeStruct((NE, wo), F32)]
    if first:
        out_specs.append(pl.BlockSpec((BE, 16), rev))
        out_shape.append(jax.ShapeDtypeStruct((NE, 16), F32))
    res = pl.pallas_call(
        functools.partial(_conv_kernel, h1=h1, h2=h2, wo=wo, first=first),
        grid=(nb,),
        in_specs=in_specs,
        out_specs=out_specs,
        out_shape=out_shape,
        scratch_shapes=[pltpu.VMEM((8, h2), F32), pltpu.SMEM((1,), I32)],
    )(*args)
    return res if first else (res[0], None)


def _mm_fused(xs, ws, masks, bias, relu, scale, bc8=None, bm=BM):
    mp = xs[0].shape[0]
    n = ws[0].shape[1]
    nx = len(xs)
    hasm = [m is not None for m in masks]

    def body(*refs):
        p = 0
        xr = refs[p:p + nx]; p += nx
        mr = []
        for j in range(nx):
            if hasm[j]:
                mr.append(refs[p]); p += 1
            else:
                mr.append(None)
        wr = refs[p:p + nx]; p += nx
        br = refs[p]; p += 1
        if bc8 is not None:
            batr, c8r = refs[p], refs[p + 1]; p += 2
        outr = refs[p]
        acc = br[...]
        for j in range(nx):
            xv = xr[j][...]
            if mr[j] is not None:
                xv = xv * mr[j][...]
            acc = acc + jnp.dot(xv, wr[j][...], preferred_element_type=F32)
        if bc8 is not None:
            oh = (batr[...] == lax.broadcasted_iota(I32, (bm, NG), 1)
                  ).astype(F32)
            acc = acc + jnp.dot(oh, c8r[...], preferred_element_type=F32)
        if relu:
            acc = jnp.maximum(acc, 0.)
        outr[...] = acc * scale

    blk = lambda i: (i, 0)
    fix = lambda i: (0, 0)
    in_specs = [pl.BlockSpec((bm, x.shape[1]), blk) for x in xs]
    args = list(xs)
    for j in range(nx):
        if hasm[j]:
            in_specs.append(pl.BlockSpec((bm, 1), blk))
            args.append(masks[j])
    for w in ws:
        in_specs.append(pl.BlockSpec((w.shape[0], n), fix))
        args.append(w)
    in_specs.append(pl.BlockSpec((1, n), fix))
    args.append((bias if bias is not None
                 else jnp.zeros((n,), F32)).reshape(1, -1))
    if bc8 is not None:
        in_specs.append(pl.BlockSpec((bm, 1), blk))
        in_specs.append(pl.BlockSpec((NG, n), fix))
        args += [bc8[0], bc8[1]]
    return pl.pallas_call(
        body,
        grid=(mp // bm,),
        in_specs=in_specs,
        out_specs=pl.BlockSpec((bm, n), blk),
        out_shape=jax.ShapeDtypeStruct((mp, n), F32),
    )(*args)


def _mm(x, w, b=None, relu=False, scale=1.0, bm=BM):
    return _mm_fused([x], [w], [None], b, relu, scale, bm=bm)


def _pool_kernel(x, bat, out, acc):
    i = pl.program_id(0)

    @pl.when(i == 0)
    def _():
        acc[...] = jnp.zeros_like(acc)

    xv = x[...]
    bv = bat[...]
    for g in range(NG):
        sel = jnp.where(bv == g, xv, 0.)
        acc[g:g + 1, :] = jnp.maximum(acc[g:g + 1, :],
                                      jnp.max(sel, axis=0, keepdims=True))

    @pl.when(i == pl.num_programs(0) - 1)
    def _():
        out[...] = acc[...]


def _pool(x4, batp):
    n = x4.shape[1]
    blk = lambda i: (i, 0)
    return pl.pallas_call(
        _pool_kernel,
        grid=(NP // BM,),
        in_specs=[pl.BlockSpec((BM, n), blk), pl.BlockSpec((BM, 1), blk)],
        out_specs=pl.BlockSpec((NG, n), lambda i: (0, 0)),
        out_shape=jax.ShapeDtypeStruct((NG, n), F32),
        scratch_shapes=[pltpu.VMEM((NG, n), F32)],
    )(x4, batp)


def _head_kernel(m0, m1, allo, aggro):
    def l2n(v):
        nrm = jnp.sqrt(jnp.sum(v * v, axis=1, keepdims=True))
        return v / jnp.maximum(nrm, 1e-12)

    a = l2n(m0[...])
    b = l2n(m1[...])
    allo[...] = jnp.concatenate([a, b], axis=1)
    aggro[...] = l2n(jnp.maximum(a, b))


def _head(m0, m1):
    blk = lambda i: (i, 0)
    return pl.pallas_call(
        _head_kernel,
        grid=(NP // BM,),
        in_specs=[pl.BlockSpec((BM, 32), blk), pl.BlockSpec((BM, 32), blk)],
        out_specs=[pl.BlockSpec((BM, 64), blk), pl.BlockSpec((BM, 32), blk)],
        out_shape=[jax.ShapeDtypeStruct((NP, 64), F32),
                   jax.ShapeDtypeStruct((NP, 32), F32)],
    )(m0, m1)


def _edge_prep(ei):
    src, dst = ei[0].astype(I32), ei[1].astype(I32)
    q = jnp.sort((dst << 14) | src)
    dst_s = q >> 14
    src_s = q & 0x3FFF
    nid = jnp.arange(NP, dtype=I32)
    r = jnp.searchsorted(dst_s, nid, side='right').astype(I32)
    lo = jnp.concatenate([jnp.zeros((1,), I32), r[:-1]])
    mask = (r > lo).astype(F32).reshape(NP, 1)
    seg = jnp.minimum(lo, NE - 1)
    return dst_s, src_s, dst_s.reshape(NE, 1), seg, mask


def _gcu(p, x, es_t, es_g, pdiffs, pos16):
    halves = []
    new_pd = [None, None]
    for ix, (es, pos_key, nn_key) in enumerate((
            (es_t, 'tpl_pos', 'tpl_nn'), (es_g, 'geo_pos', 'geo_nn'))):
        dst_s, src_s, dst2, seg, mask = es
        wp0, bp0 = p[pos_key][0]
        (w1, b1), (w2, b2) = p[nn_key]
        cin = (w1.shape[0] - 16) // 2
        wa, wb = w1[:cin], w1[cin:2 * cin]
        wc = w1[2 * cin:]
        kp = x.shape[1]
        h1 = wc.shape[1]
        wt = max(h1, 128)
        first = pdiffs is None
        if first:
            u = jnp.concatenate(
                [_mm(x, _pad_rows(wa - wb, kp)), -pos16,
                 jnp.zeros((NP, wt - h1 - 16), F32)], axis=1)
            v = jnp.concatenate(
                [_mm(x, _pad_rows(wb, kp)), pos16,
                 jnp.zeros((NP, wt - h1 - 16), F32)], axis=1)
        else:
            u = _mm(x, _pad_cols(_pad_rows(wa - wb, kp), wt))
            v = _mm(x, _pad_cols(_pad_rows(wb, kp), wt))
        g = _sc_gather_add(u, dst_s, v, src_s)
        pdin = None if first else pdiffs[ix]
        sm, pd_out = _conv_fused(g, pdin, dst2, _pad_rows(wp0, 16),
                                 bp0, wc, b1, w2, b2, first)
        new_pd[ix] = pdiffs[ix] if pdiffs is not None else pd_out
        halves.append((sm, seg, mask))
    (sm_t, seg_t, mask_t), (sm_g, seg_g, mask_g) = halves
    rt, rg = _sc_gather_multi([(sm_t, seg_t), (sm_g, seg_g)])
    wm, bm_ = p['mlp'][0]
    h2 = wm.shape[0] // 2
    out = _mm_fused([rt[:, :h2], rg[:, :h2]], [wm[:h2], wm[h2:]],
                    [mask_t, mask_g], bm_, True, BN_S)
    return out, tuple(new_pd)


def _gcnrig(p, posp, featp, es_t, es_g, batp, pdiffs, pos16):
    x1, pdiffs = _gcu(p['gcu_1'], _pad_cols(featp, max(8, featp.shape[1])),
                      es_t, es_g, pdiffs, pos16)
    x2, _ = _gcu(p['gcu_2'], x1, es_t, es_g, pdiffs, pos16)
    x3, _ = _gcu(p['gcu_3'], x2, es_t, es_g, pdiffs, pos16)
    wg, bg = p['mlp_glb'][0]
    x4 = _mm_fused([x1, x2, x3], [wg[:64], wg[64:320], wg[320:]],
                   [None] * 3, bg, True, BN_S)
    pool8 = _pool(x4, batp)
    (wt1, bt1), (wt2, bt2) = p['mlp_transform']
    f = featp.shape[1]
    c8 = _mm(pool8, wt1[:1024], bm=NG)
    pf = jnp.concatenate([posp, featp], axis=1)
    fp = ((3 + f + 7) // 8) * 8
    wpf = _pad_rows(wt1[1024:1027 + f], fp)
    h1 = _mm_fused([_pad_cols(pf, fp), x1, x2, x3],
                   [wpf, wt1[1027 + f:1091 + f], wt1[1091 + f:1347 + f],
                    wt1[1347 + f:]],
                   [None] * 4, bt1, True, BN_S, bc8=(batp, c8))
    h2 = _mm(h1, wt2, bt2, relu=True, scale=BN_S)
    wf, bf = p['final']
    nout = ((wf.shape[1] + 7) // 8) * 8
    out = _mm(h2, _pad_cols(wf, nout),
              _pad_cols(bf.reshape(1, -1), nout)[0])
    return out, pdiffs


def kernel(pos, input_flow, tpl_edge_index, geo_edge_index, batch, params):
    posp = _pad_rows(pos, NP)
    pos16 = _pad_cols(posp, 16)
    batp = jnp.pad(batch.astype(I32), (0, NP - NN),
                   constant_values=NG).reshape(NP, 1)
    es_t = _edge_prep(tpl_edge_index)
    es_g = _edge_prep(geo_edge_index)

    mp = params['motionNet']
    ms = []
    pdiffs = None
    for t in range(NKF):
        featp = _pad_rows(input_flow[:, 3 * t:3 * t + 3], NP)
        m, pdiffs = _gcnrig(mp, posp, featp, es_t, es_g, batp, pdiffs, pos16)
        ms.append(m)
    mall, maggr = _head(ms[0][:, :32], ms[1][:, :32])
    shift, _ = _gcnrig(params['jointnet'], posp, maggr, es_t, es_g, batp,
                       pdiffs, pos16)
    motion_all = mall[:NN].reshape(NN, NKF, 32)
    motion_aggr = maggr[:NN]
    pred_shift = shift[:NN, :3]
    return (motion_all, motion_aggr, pred_shift)


# speedup vs baseline: 1.0332x; 1.0332x over previous
import functools

import jax
import jax.numpy as jnp
from jax import lax
from jax.experimental import pallas as pl
from jax.experimental.pallas import tpu as pltpu
from jax.experimental.pallas import tpu_sc as plsc

NN = 10000
NP = 10240
NE = 160000
NKF = 2
NG = 8
BN_S = 1.0 / (1.0 + 1e-5) ** 0.5
BE = 1000
BM = 1024
F32 = jnp.float32
I32 = jnp.int32


def _pad_cols(x, w):
    return jnp.pad(x, ((0, 0), (0, w - x.shape[1]))) if x.shape[1] != w else x


def _pad_rows(x, r):
    return jnp.pad(x, ((0, r - x.shape[0]), (0, 0))) if x.shape[0] != r else x


def _sc_gather_multi(specs):
    info = plsc.get_sparse_core_info()
    nc, ns = info.num_cores, info.num_subcores
    nw = nc * ns
    nsp = len(specs)
    plan = []
    for table, idx in specs:
        w = table.shape[1]
        b = idx.shape[0]
        assert b % (8 * nw) == 0 and w % 128 == 0
        nb = b // nw
        bufb = 460000 // nsp - nb * 4
        best = None
        for nbuf in (5, 4, 3, 2, 1):
            for c in range(nb, 7, -1):
                if nb % c or c % 8 or (nb // c) % nbuf:
                    continue
                if nbuf * c * w * 4 <= bufb:
                    best = (nb, c, nbuf)
                    break
            if best:
                break
        assert best, (nb, w)
        plan.append(best)

    mesh = plsc.VectorSubcoreMesh(core_axis_name="c", subcore_axis_name="s")
    out_types = [jax.ShapeDtypeStruct(idx.shape + (t.shape[1],), F32)
                 for t, idx in specs]
    scratch = []
    for (t, _), (nb, c, nbuf) in zip(specs, plan):
        scratch.append(pltpu.VMEM((nb,), I32))
        for _ in range(nbuf):
            scratch.append(pltpu.VMEM((c, t.shape[1]), F32))
            scratch.append(pltpu.SemaphoreType.DMA)
            scratch.append(pltpu.SemaphoreType.DMA)

    @functools.partial(pl.kernel, mesh=mesh, out_type=out_types,
                       scratch_types=scratch)
    def k(*refs):
        tables = refs[0:2 * nsp:2]
        idxs = refs[1:2 * nsp:2]
        outs = refs[2 * nsp:3 * nsp]
        sc = list(refs[3 * nsp:])
        wid = lax.axis_index("s") * nc + lax.axis_index("c")
        p = 0
        for j in range(nsp):
            nb, c, nbuf = plan[j]
            tab, out = tables[j], outs[j]
            idxf = sc[p]; p += 1
            rows, sg, sw = [], [], []
            for _ in range(nbuf):
                rows.append(sc[p]); sg.append(sc[p + 1]); sw.append(sc[p + 2])
                p += 3
            ngroups = (nb // c) // nbuf
            base = wid * nb
            pltpu.sync_copy(idxs[j].at[pl.ds(base, nb)], idxf)
            for b in range(nbuf):
                pltpu.async_copy(tab.at[idxf.at[pl.ds(b * c, c)]],
                                 rows[b], sg[b])

            def group(cg, _, c=c, nbuf=nbuf, ngroups=ngroups, base=base,
                      tab=tab, out=out, idxf=idxf, rows=rows, sg=sg, sw=sw):
                for b in range(nbuf):
                    loc = (cg * nbuf + b) * c
                    pltpu.make_async_copy(
                        tab.at[idxf.at[pl.ds(loc, c)]], rows[b], sg[b]).wait()
                    pltpu.async_copy(rows[b], out.at[pl.ds(base + loc, c)],
                                     sw[b])

                    @pl.when(cg < ngroups - 1)
                    def _(b=b, loc=loc):
                        pltpu.make_async_copy(
                            rows[b], out.at[pl.ds(base + loc, c)],
                            sw[b]).wait()
                        nxt = loc + nbuf * c
                        pltpu.async_copy(tab.at[idxf.at[pl.ds(nxt, c)]],
                                         rows[b], sg[b])
                return 0

            lax.fori_loop(0, ngroups, group, 0)
            for b in range(nbuf):
                pltpu.make_async_copy(rows[b], out.at[pl.ds(base, c)],
                                      sw[b]).wait()

    flat = []
    for t, idx in specs:
        flat += [t, idx]
    out = k(*flat)
    return list(out) if isinstance(out, (tuple, list)) else [out]


def _sc_gather_add(t1, i1, t2, i2):
    info = plsc.get_sparse_core_info()
    nc, ns = info.num_cores, info.num_subcores
    nw = nc * ns
    w = t1.shape[1]
    b = i1.shape[0]
    assert b % (8 * nw) == 0 and w % 128 == 0
    nb = b // nw
    bufb = 460000 - 2 * nb * 4
    best = None
    for nbuf in (5, 4, 3, 2, 1):
        for c in range(nb, 7, -1):
            if nb % c or c % 8 or (nb // c) % nbuf:
                continue
            if nbuf * 2 * c * w * 4 <= bufb:
                best = (c, nbuf)
                break
        if best:
            break
    assert best, (nb, w)
    c, nbuf = best
    ngroups = (nb // c) // nbuf

    mesh = plsc.VectorSubcoreMesh(core_axis_name="c", subcore_axis_name="s")
    scratch = [pltpu.VMEM((nb,), I32), pltpu.VMEM((nb,), I32)]
    for _ in range(nbuf):
        scratch.append(pltpu.VMEM((c, w), F32))
        scratch.append(pltpu.VMEM((c, w), F32))
        scratch.append(pltpu.SemaphoreType.DMA)
        scratch.append(pltpu.SemaphoreType.DMA)

    @functools.partial(pl.kernel, mesh=mesh,
                       out_type=jax.ShapeDtypeStruct((b, w), F32),
                       scratch_types=scratch)
    def k(t1r, i1r, t2r, i2r, out, ix1, ix2, *sc):
        wid = lax.axis_index("s") * nc + lax.axis_index("c")
        base = wid * nb
        ra, rb, sg, sw = [], [], [], []
        for bi in range(nbuf):
            ra.append(sc[4 * bi]); rb.append(sc[4 * bi + 1])
            sg.append(sc[4 * bi + 2]); sw.append(sc[4 * bi + 3])
        pltpu.sync_copy(i1r.at[pl.ds(base, nb)], ix1)
        pltpu.sync_copy(i2r.at[pl.ds(base, nb)], ix2)
        for bi in range(nbuf):
            pltpu.async_copy(t1r.at[ix1.at[pl.ds(bi * c, c)]], ra[bi], sg[bi])
            pltpu.async_copy(t2r.at[ix2.at[pl.ds(bi * c, c)]], rb[bi], sg[bi])

        def group(cg, _):
            for bi in range(nbuf):
                loc = (cg * nbuf + bi) * c
                pltpu.make_async_copy(
                    t1r.at[ix1.at[pl.ds(loc, c)]], ra[bi], sg[bi]).wait()
                pltpu.make_async_copy(
                    t2r.at[ix2.at[pl.ds(loc, c)]], rb[bi], sg[bi]).wait()

                def addrow(r, _, bi=bi):
                    for jj in range(w // 16):
                        ra[bi][r, pl.ds(jj * 16, 16)] = (
                            ra[bi][r, pl.ds(jj * 16, 16)]
                            + rb[bi][r, pl.ds(jj * 16, 16)])
                    return 0

                lax.fori_loop(0, c, addrow, 0)
                pltpu.async_copy(ra[bi], out.at[pl.ds(base + loc, c)], sw[bi])

                @pl.when(cg < ngroups - 1)
                def _(bi=bi, loc=loc):
                    pltpu.make_async_copy(
                        ra[bi], out.at[pl.ds(base + loc, c)], sw[bi]).wait()
                    nxt = loc + nbuf * c
                    pltpu.async_copy(t1r.at[ix1.at[pl.ds(nxt, c)]],
                                     ra[bi], sg[bi])
                    pltpu.async_copy(t2r.at[ix2.at[pl.ds(nxt, c)]],
                                     rb[bi], sg[bi])
            return 0

        lax.fori_loop(0, ngroups, group, 0)
        for bi in range(nbuf):
            pltpu.make_async_copy(ra[bi], out.at[pl.ds(base, c)],
                                  sw[bi]).wait()

    return k(t1, i1, t2, i2)


def _conv_kernel(*refs, h1, h2, wo, first):
    if first:
        g, dst, wp, bp, wc, b1, w2, b2, out, pdo, crow, cdst = refs
        pdiff = g[:, 32:48]
    else:
        g, pdr, dst, wp, bp, wc, b1, w2, b2, out, crow, cdst = refs
        pdiff = pdr[...]
    i = pl.program_id(0)

    @pl.when(i == 0)
    def _():
        cdst[0] = -1

    pf = jnp.maximum(jnp.dot(pdiff, wp[...],
                             preferred_element_type=F32) + bp[...], 0.) * BN_S
    l1 = g[:, :h1] + jnp.dot(pf, wc[...],
                             preferred_element_type=F32) + b1[...]
    l1 = jnp.maximum(l1, 0.) * BN_S
    m = jnp.maximum(jnp.dot(l1, w2[...],
                            preferred_element_type=F32) + b2[...], 0.) * BN_S
    d = dst[...]
    s = 1
    while s < BE:
        dsh = jnp.concatenate([d[s:], jnp.full((s, 1), -2, I32)], axis=0)
        msh = jnp.concatenate([m[s:], jnp.zeros((s, h2), F32)], axis=0)
        m = jnp.where(dsh == d, jnp.maximum(m, msh), m)
        s *= 2
    m = jnp.where(d == cdst[0], jnp.maximum(m, crow[0:1, :]), m)
    cdst[0] = d[0, 0]
    crow[0:1, :] = m[0:1, :]
    if wo > h2:
        out[...] = jnp.concatenate([m, jnp.zeros((BE, wo - h2), F32)], axis=1)
    else:
        out[...] = m
    if first:
        pdo[...] = pdiff


def _conv_fused(g, pdiff, dst2, wp, bp, wc, b1, w2, b2, first):
    wt = g.shape[1]
    h1 = wc.shape[1]
    h2 = w2.shape[1]
    wo = max(h2, 128)
    nb = NE // BE
    rev = lambda i: (nb - 1 - i, 0)
    fix = lambda i: (0, 0)
    in_specs = [pl.BlockSpec((BE, wt), rev)]
    args = [g]
    if not first:
        in_specs.append(pl.BlockSpec((BE, 16), rev))
        args.append(pdiff)
    in_specs += [
        pl.BlockSpec((BE, 1), rev),
        pl.BlockSpec((16, 16), fix),
        pl.BlockSpec((1, 16), fix),
        pl.BlockSpec((16, h1), fix),
        pl.BlockSpec((1, h1), fix),
        pl.BlockSpec((h1, h2), fix),
        pl.BlockSpec((1, h2), fix),
    ]
    args += [dst2, wp, bp.reshape(1, -1), wc, b1.reshape(1, -1), w2,
             b2.reshape(1, -1)]
    out_specs = [pl.BlockSpec((BE, wo), rev)]
    out_shape = [jax.ShapeDtypeStruct((NE, wo), F32)]
    if first:
        out_specs.append(pl.BlockSpec((BE, 16), rev))
        out_shape.append(jax.ShapeDtypeStruct((NE, 16), F32))
    res = pl.pallas_call(
        functools.partial(_conv_kernel, h1=h1, h2=h2, wo=wo, first=first),
        grid=(nb,),
        in_specs=in_specs,
        out_specs=out_specs,
        out_shape=out_shape,
        scratch_shapes=[pltpu.VMEM((8, h2), F32), pltpu.SMEM((1,), I32)],
    )(*args)
    return res if first else (res[0], None)


def _mm_fused(xs, ws, masks, bias, relu, scale, bc8=None, bm=BM):
    mp = xs[0].shape[0]
    n = ws[0].shape[1]
    nx = len(xs)
    hasm = [m is not None for m in masks]

    def body(*refs):
        p = 0
        xr = refs[p:p + nx]; p += nx
        mr = []
        for j in range(nx):
            if hasm[j]:
                mr.append(refs[p]); p += 1
            else:
                mr.append(None)
        wr = refs[p:p + nx]; p += nx
        br = refs[p]; p += 1
        if bc8 is not None:
            batr, c8r = refs[p], refs[p + 1]; p += 2
        outr = refs[p]
        acc = br[...]
        for j in range(nx):
            xv = xr[j][...]
            if mr[j] is not None:
                xv = xv * mr[j][...]
            acc = acc + jnp.dot(xv, wr[j][...], preferred_element_type=F32)
        if bc8 is not None:
            oh = (batr[...] == lax.broadcasted_iota(I32, (bm, NG), 1)
                  ).astype(F32)
            acc = acc + jnp.dot(oh, c8r[...], preferred_element_type=F32)
        if relu:
            acc = jnp.maximum(acc, 0.)
        outr[...] = acc * scale

    blk = lambda i: (i, 0)
    fix = lambda i: (0, 0)
    in_specs = [pl.BlockSpec((bm, x.shape[1]), blk) for x in xs]
    args = list(xs)
    for j in range(nx):
        if hasm[j]:
            in_specs.append(pl.BlockSpec((bm, 1), blk))
            args.append(masks[j])
    for w in ws:
        in_specs.append(pl.BlockSpec((w.shape[0], n), fix))
        args.append(w)
    in_specs.append(pl.BlockSpec((1, n), fix))
    args.append((bias if bias is not None
                 else jnp.zeros((n,), F32)).reshape(1, -1))
    if bc8 is not None:
        in_specs.append(pl.BlockSpec((bm, 1), blk))
        in_specs.append(pl.BlockSpec((NG, n), fix))
        args += [bc8[0], bc8[1]]
    return pl.pallas_call(
        body,
        grid=(mp // bm,),
        in_specs=in_specs,
        out_specs=pl.BlockSpec((bm, n), blk),
        out_shape=jax.ShapeDtypeStruct((mp, n), F32),
    )(*args)


def _mm(x, w, b=None, relu=False, scale=1.0, bm=BM):
    return _mm_fused([x], [w], [None], b, relu, scale, bm=bm)


def _pool_kernel(x, bat, out, acc):
    i = pl.program_id(0)

    @pl.when(i == 0)
    def _():
        acc[...] = jnp.zeros_like(acc)

    xv = x[...]
    bv = bat[...]
    for g in range(NG):
        sel = jnp.where(bv == g, xv, 0.)
        acc[g:g + 1, :] = jnp.maximum(acc[g:g + 1, :],
                                      jnp.max(sel, axis=0, keepdims=True))

    @pl.when(i == pl.num_programs(0) - 1)
    def _():
        out[...] = acc[...]


def _pool(x4, batp):
    n = x4.shape[1]
    blk = lambda i: (i, 0)
    return pl.pallas_call(
        _pool_kernel,
        grid=(NP // BM,),
        in_specs=[pl.BlockSpec((BM, n), blk), pl.BlockSpec((BM, 1), blk)],
        out_specs=pl.BlockSpec((NG, n), lambda i: (0, 0)),
        out_shape=jax.ShapeDtypeStruct((NG, n), F32),
        scratch_shapes=[pltpu.VMEM((NG, n), F32)],
    )(x4, batp)


def _head_kernel(m0, m1, allo, aggro):
    def l2n(v):
        nrm = jnp.sqrt(jnp.sum(v * v, axis=1, keepdims=True))
        return v / jnp.maximum(nrm, 1e-12)

    a = l2n(m0[...])
    b = l2n(m1[...])
    allo[...] = jnp.concatenate([a, b], axis=1)
    aggro[...] = l2n(jnp.maximum(a, b))


def _head(m0, m1):
    blk = lambda i: (i, 0)
    return pl.pallas_call(
        _head_kernel,
        grid=(NP // BM,),
        in_specs=[pl.BlockSpec((BM, 32), blk), pl.BlockSpec((BM, 32), blk)],
        out_specs=[pl.BlockSpec((BM, 64), blk), pl.BlockSpec((BM, 32), blk)],
        out_shape=[jax.ShapeDtypeStruct((NP, 64), F32),
                   jax.ShapeDtypeStruct((NP, 32), F32)],
    )(m0, m1)


def _edge_prep(ei):
    src, dst = ei[0].astype(I32), ei[1].astype(I32)
    q = jnp.sort((dst << 14) | src)
    dst_s = q >> 14
    src_s = q & 0x3FFF
    nid = jnp.arange(NP, dtype=I32)
    r = jnp.searchsorted(dst_s, nid, side='right').astype(I32)
    lo = jnp.concatenate([jnp.zeros((1,), I32), r[:-1]])
    mask = (r > lo).astype(F32).reshape(NP, 1)
    seg = jnp.minimum(lo, NE - 1)
    return dst_s, src_s, dst_s.reshape(NE, 1), seg, mask


def _gcu(p, x, es_t, es_g, pdiffs, pos16):
    halves = []
    new_pd = [None, None]
    for ix, (es, pos_key, nn_key) in enumerate((
            (es_t, 'tpl_pos', 'tpl_nn'), (es_g, 'geo_pos', 'geo_nn'))):
        dst_s, src_s, dst2, seg, mask = es
        wp0, bp0 = p[pos_key][0]
        (w1, b1), (w2, b2) = p[nn_key]
        cin = (w1.shape[0] - 16) // 2
        wa, wb = w1[:cin], w1[cin:2 * cin]
        wc = w1[2 * cin:]
        kp = x.shape[1]
        h1 = wc.shape[1]
        wt = max(h1, 128)
        first = pdiffs is None
        if first:
            u = jnp.concatenate(
                [_mm(x, _pad_rows(wa - wb, kp)), -pos16,
                 jnp.zeros((NP, wt - h1 - 16), F32)], axis=1)
            v = jnp.concatenate(
                [_mm(x, _pad_rows(wb, kp)), pos16,
                 jnp.zeros((NP, wt - h1 - 16), F32)], axis=1)
        else:
            u = _mm(x, _pad_cols(_pad_rows(wa - wb, kp), wt))
            v = _mm(x, _pad_cols(_pad_rows(wb, kp), wt))
        g = _sc_gather_add(u, dst_s, v, src_s)
        pdin = None if first else pdiffs[ix]
        sm, pd_out = _conv_fused(g, pdin, dst2, _pad_rows(wp0, 16),
                                 bp0, wc, b1, w2, b2, first)
        new_pd[ix] = pdiffs[ix] if pdiffs is not None else pd_out
        halves.append((sm, seg, mask))
    (sm_t, seg_t, mask_t), (sm_g, seg_g, mask_g) = halves
    rt, rg = _sc_gather_multi([(sm_t, seg_t), (sm_g, seg_g)])
    wm, bm_ = p['mlp'][0]
    h2 = wm.shape[0] // 2
    out = _mm_fused([rt[:, :h2], rg[:, :h2]], [wm[:h2], wm[h2:]],
                    [mask_t, mask_g], bm_, True, BN_S)
    return out, tuple(new_pd)


def _gcnrig(p, posp, featp, es_t, es_g, batp, pdiffs, pos16):
    x1, pdiffs = _gcu(p['gcu_1'], _pad_cols(featp, max(8, featp.shape[1])),
                      es_t, es_g, pdiffs, pos16)
    x2, _ = _gcu(p['gcu_2'], x1, es_t, es_g, pdiffs, pos16)
    x3, _ = _gcu(p['gcu_3'], x2, es_t, es_g, pdiffs, pos16)
    wg, bg = p['mlp_glb'][0]
    x4 = _mm_fused([x1, x2, x3], [wg[:64], wg[64:320], wg[320:]],
                   [None] * 3, bg, True, BN_S)
    pool8 = _pool(x4, batp)
    (wt1, bt1), (wt2, bt2) = p['mlp_transform']
    f = featp.shape[1]
    c8 = _mm(pool8, wt1[:1024], bm=NG)
    pf = jnp.concatenate([posp, featp], axis=1)
    fp = ((3 + f + 7) // 8) * 8
    wpf = _pad_rows(wt1[1024:1027 + f], fp)
    h1 = _mm_fused([_pad_cols(pf, fp), x1, x2, x3],
                   [wpf, wt1[1027 + f:1091 + f], wt1[1091 + f:1347 + f],
                    wt1[1347 + f:]],
                   [None] * 4, bt1, True, BN_S, bc8=(batp, c8))
    h2 = _mm(h1, wt2, bt2, relu=True, scale=BN_S)
    wf, bf = p['final']
    nout = ((wf.shape[1] + 7) // 8) * 8
    out = _mm(h2, _pad_cols(wf, nout),
              _pad_cols(bf.reshape(1, -1), nout)[0])
    return out, pdiffs


def kernel(pos, input_flow, tpl_edge_index, geo_edge_index, batch, params):
    posp = _pad_rows(pos, NP)
    pos16 = _pad_cols(posp, 16)
    batp = jnp.pad(batch.astype(I32), (0, NP - NN),
                   constant_values=NG).reshape(NP, 1)
    es_t = _edge_prep(tpl_edge_index)
    es_g = _edge_prep(geo_edge_index)

    mp = params['motionNet']
    ms = []
    pdiffs = None
    for t in range(NKF):
        featp = _pad_rows(input_flow[:, 3 * t:3 * t + 3], NP)
        m, pdiffs = _gcnrig(mp, posp, featp, es_t, es_g, batp, pdiffs, pos16)
        ms.append(m)
    mall, maggr = _head(ms[0][:, :32], ms[1][:, :32])
    shift, _ = _gcnrig(params['jointnet'], posp, maggr, es_t, es_g, batp,
                       pdiffs, pos16)
    motion_all = mall[:NN].reshape(NN, NKF, 32)
    motion_aggr = maggr[:NN]
    pred_shift = shift[:NN, :3]
    return (motion_all, motion_aggr, pred_shift)
